# baseline (speedup 1.0000x reference)
import jax
import jax.numpy as jnp
from jax.experimental import pallas as pl

NUM_OUTPUT = 12
H = 100
AG = 130
VOICES = 5
GP = 128
CP = 16
B_BLK = 2048


def _poly_body(x_ref, wx_ref, wsb_ref, bias_ref, wlin_ref, blin_ref,
               g0_ref, g1_ref, g2_ref, g3_ref, g4_ref,
               sampled_ref, probs_ref):
    base = jax.lax.dot_general(
        wx_ref[...], x_ref[...],
        dimension_numbers=(((1,), (1,)), ((), ())),
        preferred_element_type=jnp.float32) + bias_ref[...]
    blk = base.shape[1]
    sampled = jnp.zeros((CP, blk), jnp.float32)
    banned = jnp.zeros((CP, blk), jnp.float32)
    sample_p = jnp.zeros((CP, blk), jnp.float32)
    ban_p = jnp.zeros((CP, blk), jnp.float32)
    row = jax.lax.broadcasted_iota(jnp.int32, (2 * CP, blk), 0)
    half = jax.lax.broadcasted_iota(jnp.int32, (CP, blk), 0)
    live = (half < NUM_OUTPUT).astype(jnp.float32)
    g_refs = (g0_ref, g1_ref, g2_ref, g3_ref, g4_ref)
    for v in range(VOICES):
        sb = jnp.concatenate([sampled, banned], axis=0)
        gates = base + jnp.dot(wsb_ref[...], sb,
                               preferred_element_type=jnp.float32)
        i = jax.nn.sigmoid(gates[0:H, :])
        g = jnp.tanh(gates[GP:GP + H, :])
        o = jax.nn.sigmoid(gates[2 * GP:2 * GP + H, :])
        h = o * jnp.tanh(i * g)
        out = jnp.dot(wlin_ref[...], h,
                      preferred_element_type=jnp.float32) + blin_ref[...]
        cm = live * (1.0 - sampled) * (1.0 - banned)
        coeff = jnp.concatenate([cm, cm], axis=0)
        p = coeff * jnp.exp(out)
        p = p / jnp.sum(p, axis=0, keepdims=True)
        logits = jnp.where(p > 0, jnp.log(jnp.maximum(p, 1e-30)), -1e9)
        gz = jnp.transpose(g_refs[v][...])
        gz = jnp.concatenate([
            gz[0:NUM_OUTPUT], jnp.zeros((CP - NUM_OUTPUT, blk), jnp.float32),
            gz[NUM_OUTPUT:], jnp.zeros((CP - NUM_OUTPUT, blk), jnp.float32),
        ], axis=0)
        z = logits + gz
        m = jnp.max(z, axis=0, keepdims=True)
        idx = jnp.min(jnp.where(z == m, row, 2 * CP), axis=0, keepdims=True)
        onehot = (row == idx).astype(jnp.float32)
        note = onehot[0:CP, :]
        ban = onehot[CP:, :]
        sample_p = sample_p + note * p[0:CP, :]
        ban_p = ban_p + ban * p[CP:, :]
        sampled = jnp.minimum(sampled + note, 1.0)
        banned = jnp.minimum(banned + ban, 1.0)
    sampled_ref[...] = sampled
    probs_ref[...] = jnp.concatenate([sample_p, ban_p], axis=0)


def kernel(x, W_ih, W_hh, b_ih, b_hh, W_lin, b_lin):
    del W_hh
    B = x.shape[1]
    xf = x[0]
    Wp = jnp.zeros((3 * GP, AG + 2 * NUM_OUTPUT), jnp.float32)
    bias = b_ih + b_hh
    bp = jnp.zeros((3 * GP,), jnp.float32)
    for slot, (lo, hi) in enumerate(((0, H), (2 * H, 3 * H), (3 * H, 4 * H))):
        Wp = Wp.at[slot * GP:slot * GP + H].set(W_ih[lo:hi])
        bp = bp.at[slot * GP:slot * GP + H].set(bias[lo:hi])
    wx = Wp[:, :AG]
    wsb = jnp.zeros((3 * GP, 2 * CP), jnp.float32)
    wsb = wsb.at[:, 0:NUM_OUTPUT].set(Wp[:, AG:AG + NUM_OUTPUT])
    wsb = wsb.at[:, CP:CP + NUM_OUTPUT].set(Wp[:, AG + NUM_OUTPUT:])
    wlp = jnp.zeros((2 * CP, H), jnp.float32)
    wlp = wlp.at[0:NUM_OUTPUT].set(W_lin[:NUM_OUTPUT])
    wlp = wlp.at[CP:CP + NUM_OUTPUT].set(W_lin[NUM_OUTPUT:])
    blp = jnp.zeros((2 * CP,), jnp.float32)
    blp = blp.at[0:NUM_OUTPUT].set(b_lin[:NUM_OUTPUT])
    blp = blp.at[CP:CP + NUM_OUTPUT].set(b_lin[NUM_OUTPUT:])
    skey = jax.random.key(42)
    Gs = [jax.random.gumbel(jax.random.fold_in(skey, v), (B, 2 * NUM_OUTPUT),
                            jnp.float32) for v in range(VOICES)]

    grid = (B // B_BLK,)
    gspec = pl.BlockSpec((B_BLK, 2 * NUM_OUTPUT), lambda i: (i, 0))
    sampled_t, probs_t = pl.pallas_call(
        _poly_body,
        grid=grid,
        in_specs=[
            pl.BlockSpec((B_BLK, AG), lambda i: (i, 0)),
            pl.BlockSpec((3 * GP, AG), lambda i: (0, 0)),
            pl.BlockSpec((3 * GP, 2 * CP), lambda i: (0, 0)),
            pl.BlockSpec((3 * GP, 1), lambda i: (0, 0)),
            pl.BlockSpec((2 * CP, H), lambda i: (0, 0)),
            pl.BlockSpec((2 * CP, 1), lambda i: (0, 0)),
            gspec, gspec, gspec, gspec, gspec,
        ],
        out_specs=[
            pl.BlockSpec((CP, B_BLK), lambda i: (0, i)),
            pl.BlockSpec((2 * CP, B_BLK), lambda i: (0, i)),
        ],
        out_shape=[
            jax.ShapeDtypeStruct((CP, B), jnp.float32),
            jax.ShapeDtypeStruct((2 * CP, B), jnp.float32),
        ],
    )(xf, wx, wsb, bp[:, None], wlp, blp[:, None], *Gs)
    sampled = sampled_t[:NUM_OUTPUT].T[None]
    probs = jnp.concatenate(
        [probs_t[0:NUM_OUTPUT], probs_t[CP:CP + NUM_OUTPUT]], axis=0).T[None]
    return (sampled, probs)


# speedup vs baseline: 1.9516x; 1.9516x over previous
import jax
import jax.numpy as jnp
import numpy as np
from jax.experimental import pallas as pl

NUM_OUTPUT = 12
H = 100
AG = 130
VOICES = 5
CP = 16
NC = 2 * CP
B_BLK = 2048
TINY = float(np.finfo(np.float32).tiny)

_M32 = 0xFFFFFFFF
_ROT_A = (13, 15, 26, 6)
_ROT_B = (17, 29, 16, 24)


def _tf2x32_py(k1, k2, x1, x2):
    ks = (k1, k2, (k1 ^ k2 ^ 0x1BD11BDA) & _M32)
    x = [(x1 + ks[0]) & _M32, (x2 + ks[1]) & _M32]
    sched = ((_ROT_A, ks[1], ks[2], 1), (_ROT_B, ks[2], ks[0], 2),
             (_ROT_A, ks[0], ks[1], 3), (_ROT_B, ks[1], ks[2], 4),
             (_ROT_A, ks[2], ks[0], 5))
    for rots, a0, a1, i in sched:
        for r in rots:
            x[0] = (x[0] + x[1]) & _M32
            x[1] = ((x[1] << r) | (x[1] >> (32 - r))) & _M32
            x[1] = x[0] ^ x[1]
        x[0] = (x[0] + a0) & _M32
        x[1] = (x[1] + a1 + i) & _M32
    return x[0], x[1]


_VOICE_KEYS = tuple(_tf2x32_py(0, 42, 0, v) for v in range(VOICES))


def _tf2x32_vec(k1, k2, x2):
    ks = (jnp.uint32(k1), jnp.uint32(k2),
          jnp.uint32((k1 ^ k2 ^ 0x1BD11BDA) & _M32))
    x0 = jnp.full(x2.shape, ks[0], jnp.uint32)
    x1 = x2 + ks[1]
    sched = ((_ROT_A, ks[1], ks[2], 1), (_ROT_B, ks[2], ks[0], 2),
             (_ROT_A, ks[0], ks[1], 3), (_ROT_B, ks[1], ks[2], 4),
             (_ROT_A, ks[2], ks[0], 5))
    for rots, a0, a1, i in sched:
        for r in rots:
            x0 = x0 + x1
            x1 = (x1 << jnp.uint32(r)) | (x1 >> jnp.uint32(32 - r))
            x1 = x0 ^ x1
        x0 = x0 + a0
        x1 = x1 + a1 + jnp.uint32(i)
    return x0 ^ x1


def _poly_body(x_ref, wih_ref, bih_ref, bhh_ref, wlin_ref, blin_ref,
               sampled_ref, probs_ref):
    blk = x_ref.shape[0]
    wih = wih_ref[...]
    bias = bih_ref[...] + bhh_ref[...]
    wi, wg, wo = wih[0:H], wih[2 * H:3 * H], wih[3 * H:4 * H]
    bi, bg, bo = bias[0:H], bias[2 * H:3 * H], bias[3 * H:4 * H]
    dn = (((1,), (1,)), ((), ()))
    x = x_ref[...]
    base_i = jax.lax.dot_general(wi[:, :AG], x, dn,
                                 preferred_element_type=jnp.float32) + bi
    base_g = jax.lax.dot_general(wg[:, :AG], x, dn,
                                 preferred_element_type=jnp.float32) + bg
    base_o = jax.lax.dot_general(wo[:, :AG], x, dn,
                                 preferred_element_type=jnp.float32) + bo
    zc = jnp.zeros((H, CP - NUM_OUTPUT), jnp.float32)
    wsb_i = jnp.concatenate([wi[:, AG:AG + NUM_OUTPUT], zc,
                             wi[:, AG + NUM_OUTPUT:], zc], axis=1)
    wsb_g = jnp.concatenate([wg[:, AG:AG + NUM_OUTPUT], zc,
                             wg[:, AG + NUM_OUTPUT:], zc], axis=1)
    wsb_o = jnp.concatenate([wo[:, AG:AG + NUM_OUTPUT], zc,
                             wo[:, AG + NUM_OUTPUT:], zc], axis=1)
    wl = wlin_ref[...]
    bl = blin_ref[...]
    zr = jnp.zeros((CP - NUM_OUTPUT, H), jnp.float32)
    zb = jnp.zeros((CP - NUM_OUTPUT, 1), jnp.float32)
    wl32 = jnp.concatenate([wl[0:NUM_OUTPUT], zr, wl[NUM_OUTPUT:], zr], axis=0)
    bl32 = jnp.concatenate([bl[0:NUM_OUTPUT], zb, bl[NUM_OUTPUT:], zb], axis=0)

    sampled = jnp.zeros((CP, blk), jnp.float32)
    banned = jnp.zeros((CP, blk), jnp.float32)
    sample_p = jnp.zeros((CP, blk), jnp.float32)
    ban_p = jnp.zeros((CP, blk), jnp.float32)
    row = jax.lax.broadcasted_iota(jnp.int32, (NC, blk), 0)
    live = (jax.lax.broadcasted_iota(jnp.int32, (CP, blk), 0)
            < NUM_OUTPUT).astype(jnp.float32)
    urow = jax.lax.broadcasted_iota(jnp.uint32, (NC, blk), 0)
    ulane = jax.lax.broadcasted_iota(jnp.uint32, (NC, blk), 1)
    ceff = jnp.where(urow >= CP, urow - jnp.uint32(CP - NUM_OUTPUT), urow)
    b0 = (pl.program_id(0) * B_BLK).astype(jnp.uint32)
    flat = (b0 + ulane) * jnp.uint32(2 * NUM_OUTPUT) + ceff

    for v in range(VOICES):
        sb = jnp.concatenate([sampled, banned], axis=0)
        gi = base_i + jnp.dot(wsb_i, sb, preferred_element_type=jnp.float32)
        gg = base_g + jnp.dot(wsb_g, sb, preferred_element_type=jnp.float32)
        go = base_o + jnp.dot(wsb_o, sb, preferred_element_type=jnp.float32)
        h = jax.nn.sigmoid(go) * jnp.tanh(jax.nn.sigmoid(gi) * jnp.tanh(gg))
        out = jnp.dot(wl32, h, preferred_element_type=jnp.float32) + bl32
        cm = live * (1.0 - sampled) * (1.0 - banned)
        coeff = jnp.concatenate([cm, cm], axis=0)
        p = coeff * jnp.exp(out)
        p = p / jnp.sum(p, axis=0, keepdims=True)
        logits = jnp.where(p > 0, jnp.log(jnp.maximum(p, 1e-30)), -1e9)
        bits = _tf2x32_vec(*_VOICE_KEYS[v], flat)
        fb = (bits >> jnp.uint32(9)) | jnp.uint32(0x3F800000)
        u = jax.lax.bitcast_convert_type(fb, jnp.float32) - 1.0
        un = jnp.maximum(jnp.float32(TINY), u + jnp.float32(TINY))
        gz = -jnp.log(-jnp.log(un))
        z = logits + gz
        m = jnp.max(z, axis=0, keepdims=True)
        idx = jnp.min(jnp.where(z == m, row, NC), axis=0, keepdims=True)
        onehot = (row == idx).astype(jnp.float32)
        note = onehot[0:CP, :]
        ban = onehot[CP:, :]
        sample_p = sample_p + note * p[0:CP, :]
        ban_p = ban_p + ban * p[CP:, :]
        sampled = jnp.minimum(sampled + note, 1.0)
        banned = jnp.minimum(banned + ban, 1.0)

    sampled_ref[...] = jnp.transpose(sampled[0:NUM_OUTPUT])
    pn = jnp.transpose(sample_p[0:NUM_OUTPUT])
    pb = jnp.transpose(ban_p[0:NUM_OUTPUT])
    probs_ref[...] = jnp.concatenate([pn, pb], axis=1)


def kernel(x, W_ih, W_hh, b_ih, b_hh, W_lin, b_lin):
    del W_hh
    B = x.shape[1]
    grid = (B // B_BLK,)
    sampled, probs = pl.pallas_call(
        _poly_body,
        grid=grid,
        in_specs=[
            pl.BlockSpec((B_BLK, AG), lambda i: (i, 0)),
            pl.BlockSpec((4 * H, AG + 2 * NUM_OUTPUT), lambda i: (0, 0)),
            pl.BlockSpec((4 * H, 1), lambda i: (0, 0)),
            pl.BlockSpec((4 * H, 1), lambda i: (0, 0)),
            pl.BlockSpec((2 * NUM_OUTPUT, H), lambda i: (0, 0)),
            pl.BlockSpec((2 * NUM_OUTPUT, 1), lambda i: (0, 0)),
        ],
        out_specs=[
            pl.BlockSpec((B_BLK, NUM_OUTPUT), lambda i: (i, 0)),
            pl.BlockSpec((B_BLK, 2 * NUM_OUTPUT), lambda i: (i, 0)),
        ],
        out_shape=[
            jax.ShapeDtypeStruct((B, NUM_OUTPUT), jnp.float32),
            jax.ShapeDtypeStruct((B, 2 * NUM_OUTPUT), jnp.float32),
        ],
    )(x[0], W_ih, b_ih[:, None], b_hh[:, None], W_lin, b_lin[:, None])
    return (sampled[None], probs[None])


# speedup vs baseline: 2.0987x; 1.0754x over previous
import jax
import jax.numpy as jnp
import numpy as np
from jax.experimental import pallas as pl

NUM_OUTPUT = 12
H = 100
AG = 130
VOICES = 5
CP = 16
NC = 2 * CP
B_BLK = 4096
TINY = float(np.finfo(np.float32).tiny)

_M32 = 0xFFFFFFFF
_ROT_A = (13, 15, 26, 6)
_ROT_B = (17, 29, 16, 24)


def _tf2x32_py(k1, k2, x1, x2):
    ks = (k1, k2, (k1 ^ k2 ^ 0x1BD11BDA) & _M32)
    x = [(x1 + ks[0]) & _M32, (x2 + ks[1]) & _M32]
    sched = ((_ROT_A, ks[1], ks[2], 1), (_ROT_B, ks[2], ks[0], 2),
             (_ROT_A, ks[0], ks[1], 3), (_ROT_B, ks[1], ks[2], 4),
             (_ROT_A, ks[2], ks[0], 5))
    for rots, a0, a1, i in sched:
        for r in rots:
            x[0] = (x[0] + x[1]) & _M32
            x[1] = ((x[1] << r) | (x[1] >> (32 - r))) & _M32
            x[1] = x[0] ^ x[1]
        x[0] = (x[0] + a0) & _M32
        x[1] = (x[1] + a1 + i) & _M32
    return x[0], x[1]


_VOICE_KEYS = tuple(_tf2x32_py(0, 42, 0, v) for v in range(VOICES))


def _tf2x32_vec(k1, k2, x2):
    ks = (jnp.uint32(k1), jnp.uint32(k2),
          jnp.uint32((k1 ^ k2 ^ 0x1BD11BDA) & _M32))
    x0 = jnp.full(x2.shape, ks[0], jnp.uint32)
    x1 = x2 + ks[1]
    sched = ((_ROT_A, ks[1], ks[2], 1), (_ROT_B, ks[2], ks[0], 2),
             (_ROT_A, ks[0], ks[1], 3), (_ROT_B, ks[1], ks[2], 4),
             (_ROT_A, ks[2], ks[0], 5))
    for rots, a0, a1, i in sched:
        for r in rots:
            x0 = x0 + x1
            x1 = (x1 << jnp.uint32(r)) | (x1 >> jnp.uint32(32 - r))
            x1 = x0 ^ x1
        x0 = x0 + a0
        x1 = x1 + a1 + jnp.uint32(i)
    return x0 ^ x1


def _poly_body(x_ref, wih_ref, bih_ref, bhh_ref, wlin_ref, blin_ref,
               sampled_ref, probs_ref):
    blk = x_ref.shape[0]
    wih = wih_ref[...]
    bias = bih_ref[...] + bhh_ref[...]
    wi, wg, wo = wih[0:H], wih[2 * H:3 * H], wih[3 * H:4 * H]
    bi, bg, bo = bias[0:H], bias[2 * H:3 * H], bias[3 * H:4 * H]
    dn = (((1,), (1,)), ((), ()))
    x = x_ref[...]
    base_i = jax.lax.dot_general(wi[:, :AG], x, dn,
                                 preferred_element_type=jnp.float32) + bi
    base_g = jax.lax.dot_general(wg[:, :AG], x, dn,
                                 preferred_element_type=jnp.float32) + bg
    base_o = jax.lax.dot_general(wo[:, :AG], x, dn,
                                 preferred_element_type=jnp.float32) + bo
    zc = jnp.zeros((H, CP - NUM_OUTPUT), jnp.float32)
    wsb_i = jnp.concatenate([wi[:, AG:AG + NUM_OUTPUT], zc,
                             wi[:, AG + NUM_OUTPUT:], zc], axis=1)
    wsb_g = jnp.concatenate([wg[:, AG:AG + NUM_OUTPUT], zc,
                             wg[:, AG + NUM_OUTPUT:], zc], axis=1)
    wsb_o = jnp.concatenate([wo[:, AG:AG + NUM_OUTPUT], zc,
                             wo[:, AG + NUM_OUTPUT:], zc], axis=1)
    wl = wlin_ref[...]
    bl = blin_ref[...]
    zr = jnp.zeros((CP - NUM_OUTPUT, H), jnp.float32)
    zb = jnp.zeros((CP - NUM_OUTPUT, 1), jnp.float32)
    wl32 = jnp.concatenate([wl[0:NUM_OUTPUT], zr, wl[NUM_OUTPUT:], zr], axis=0)
    bl32 = jnp.concatenate([bl[0:NUM_OUTPUT], zb, bl[NUM_OUTPUT:], zb], axis=0)

    sampled = jnp.zeros((CP, blk), jnp.float32)
    banned = jnp.zeros((CP, blk), jnp.float32)
    sample_p = jnp.zeros((CP, blk), jnp.float32)
    ban_p = jnp.zeros((CP, blk), jnp.float32)
    row = jax.lax.broadcasted_iota(jnp.int32, (NC, blk), 0)
    live = (jax.lax.broadcasted_iota(jnp.int32, (CP, blk), 0)
            < NUM_OUTPUT).astype(jnp.float32)
    urow = jax.lax.broadcasted_iota(jnp.uint32, (2 * NUM_OUTPUT, blk), 0)
    ulane = jax.lax.broadcasted_iota(jnp.uint32, (2 * NUM_OUTPUT, blk), 1)
    b0 = (pl.program_id(0) * B_BLK).astype(jnp.uint32)
    flat = (b0 + ulane) * jnp.uint32(2 * NUM_OUTPUT) + urow
    zpad = jnp.zeros((CP - NUM_OUTPUT, blk), jnp.float32)

    for v in range(VOICES):
        sb = jnp.concatenate([sampled, banned], axis=0)
        gi = base_i + jnp.dot(wsb_i, sb, preferred_element_type=jnp.float32)
        gg = base_g + jnp.dot(wsb_g, sb, preferred_element_type=jnp.float32)
        go = base_o + jnp.dot(wsb_o, sb, preferred_element_type=jnp.float32)
        h = jax.nn.sigmoid(go) * jnp.tanh(jax.nn.sigmoid(gi) * jnp.tanh(gg))
        out = jnp.dot(wl32, h, preferred_element_type=jnp.float32) + bl32
        cm = live * (1.0 - sampled) * (1.0 - banned)
        coeff = jnp.concatenate([cm, cm], axis=0)
        p = coeff * jnp.exp(out)
        p = p / jnp.sum(p, axis=0, keepdims=True)
        logits = jnp.where(p > 0, jnp.log(jnp.maximum(p, 1e-30)), -1e9)
        bits = _tf2x32_vec(*_VOICE_KEYS[v], flat)
        fb = (bits >> jnp.uint32(9)) | jnp.uint32(0x3F800000)
        u = jax.lax.bitcast_convert_type(fb, jnp.float32) - 1.0
        un = jnp.maximum(jnp.float32(TINY), u + jnp.float32(TINY))
        g24 = -jnp.log(-jnp.log(un))
        gz = jnp.concatenate([g24[0:NUM_OUTPUT], zpad,
                              g24[NUM_OUTPUT:], zpad], axis=0)
        z = logits + gz
        m = jnp.max(z, axis=0, keepdims=True)
        idx = jnp.min(jnp.where(z == m, row, NC), axis=0, keepdims=True)
        onehot = (row == idx).astype(jnp.float32)
        note = onehot[0:CP, :]
        ban = onehot[CP:, :]
        sample_p = sample_p + note * p[0:CP, :]
        ban_p = ban_p + ban * p[CP:, :]
        sampled = jnp.minimum(sampled + note, 1.0)
        banned = jnp.minimum(banned + ban, 1.0)

    sampled_ref[...] = jnp.transpose(sampled[0:NUM_OUTPUT])
    pn = jnp.transpose(sample_p[0:NUM_OUTPUT])
    pb = jnp.transpose(ban_p[0:NUM_OUTPUT])
    probs_ref[...] = jnp.concatenate([pn, pb], axis=1)


def kernel(x, W_ih, W_hh, b_ih, b_hh, W_lin, b_lin):
    del W_hh
    B = x.shape[1]
    grid = (B // B_BLK,)
    sampled, probs = pl.pallas_call(
        _poly_body,
        grid=grid,
        in_specs=[
            pl.BlockSpec((B_BLK, AG), lambda i: (i, 0)),
            pl.BlockSpec((4 * H, AG + 2 * NUM_OUTPUT), lambda i: (0, 0)),
            pl.BlockSpec((4 * H, 1), lambda i: (0, 0)),
            pl.BlockSpec((4 * H, 1), lambda i: (0, 0)),
            pl.BlockSpec((2 * NUM_OUTPUT, H), lambda i: (0, 0)),
            pl.BlockSpec((2 * NUM_OUTPUT, 1), lambda i: (0, 0)),
        ],
        out_specs=[
            pl.BlockSpec((B_BLK, NUM_OUTPUT), lambda i: (i, 0)),
            pl.BlockSpec((B_BLK, 2 * NUM_OUTPUT), lambda i: (i, 0)),
        ],
        out_shape=[
            jax.ShapeDtypeStruct((B, NUM_OUTPUT), jnp.float32),
            jax.ShapeDtypeStruct((B, 2 * NUM_OUTPUT), jnp.float32),
        ],
    )(x[0], W_ih, b_ih[:, None], b_hh[:, None], W_lin, b_lin[:, None])
    return (sampled[None], probs[None])


# speedup vs baseline: 2.1009x; 1.0011x over previous
import jax
import jax.numpy as jnp
import numpy as np
from jax.experimental import pallas as pl
from jax.experimental.pallas import tpu as pltpu

NUM_OUTPUT = 12
H = 100
AG = 130
VOICES = 5
CP = 16
NC = 2 * CP
B_BLK = 4096
TINY = float(np.finfo(np.float32).tiny)

_M32 = 0xFFFFFFFF
_ROT_A = (13, 15, 26, 6)
_ROT_B = (17, 29, 16, 24)


def _tf2x32_py(k1, k2, x1, x2):
    ks = (k1, k2, (k1 ^ k2 ^ 0x1BD11BDA) & _M32)
    x = [(x1 + ks[0]) & _M32, (x2 + ks[1]) & _M32]
    sched = ((_ROT_A, ks[1], ks[2], 1), (_ROT_B, ks[2], ks[0], 2),
             (_ROT_A, ks[0], ks[1], 3), (_ROT_B, ks[1], ks[2], 4),
             (_ROT_A, ks[2], ks[0], 5))
    for rots, a0, a1, i in sched:
        for r in rots:
            x[0] = (x[0] + x[1]) & _M32
            x[1] = ((x[1] << r) | (x[1] >> (32 - r))) & _M32
            x[1] = x[0] ^ x[1]
        x[0] = (x[0] + a0) & _M32
        x[1] = (x[1] + a1 + i) & _M32
    return x[0], x[1]


_VOICE_KEYS = tuple(_tf2x32_py(0, 42, 0, v) for v in range(VOICES))


def _tf2x32_vec(k1, k2, x2):
    ks = (jnp.uint32(k1), jnp.uint32(k2),
          jnp.uint32((k1 ^ k2 ^ 0x1BD11BDA) & _M32))
    x0 = jnp.full(x2.shape, ks[0], jnp.uint32)
    x1 = x2 + ks[1]
    sched = ((_ROT_A, ks[1], ks[2], 1), (_ROT_B, ks[2], ks[0], 2),
             (_ROT_A, ks[0], ks[1], 3), (_ROT_B, ks[1], ks[2], 4),
             (_ROT_A, ks[2], ks[0], 5))
    for rots, a0, a1, i in sched:
        for r in rots:
            x0 = x0 + x1
            x1 = (x1 << jnp.uint32(r)) | (x1 >> jnp.uint32(32 - r))
            x1 = x0 ^ x1
        x0 = x0 + a0
        x1 = x1 + a1 + jnp.uint32(i)
    return x0 ^ x1


def _poly_body(x_ref, wih_ref, bih_ref, bhh_ref, wlin_ref, blin_ref,
               sampled_ref, probs_ref):
    blk = x_ref.shape[0]
    wih = wih_ref[...]
    bias = bih_ref[...] + bhh_ref[...]
    wi, wg, wo = wih[0:H], wih[2 * H:3 * H], wih[3 * H:4 * H]
    bi, bg, bo = bias[0:H], bias[2 * H:3 * H], bias[3 * H:4 * H]
    dn = (((1,), (1,)), ((), ()))
    x = x_ref[...]
    base_i = jax.lax.dot_general(wi[:, :AG], x, dn,
                                 preferred_element_type=jnp.float32) + bi
    base_g = jax.lax.dot_general(wg[:, :AG], x, dn,
                                 preferred_element_type=jnp.float32) + bg
    base_o = jax.lax.dot_general(wo[:, :AG], x, dn,
                                 preferred_element_type=jnp.float32) + bo
    zc = jnp.zeros((H, CP - NUM_OUTPUT), jnp.float32)
    wsb_i = jnp.concatenate([wi[:, AG:AG + NUM_OUTPUT], zc,
                             wi[:, AG + NUM_OUTPUT:], zc], axis=1)
    wsb_g = jnp.concatenate([wg[:, AG:AG + NUM_OUTPUT], zc,
                             wg[:, AG + NUM_OUTPUT:], zc], axis=1)
    wsb_o = jnp.concatenate([wo[:, AG:AG + NUM_OUTPUT], zc,
                             wo[:, AG + NUM_OUTPUT:], zc], axis=1)
    wl = wlin_ref[...]
    bl = blin_ref[...]
    zr = jnp.zeros((CP - NUM_OUTPUT, H), jnp.float32)
    zb = jnp.zeros((CP - NUM_OUTPUT, 1), jnp.float32)
    wl32 = jnp.concatenate([wl[0:NUM_OUTPUT], zr, wl[NUM_OUTPUT:], zr], axis=0)
    bl32 = jnp.concatenate([bl[0:NUM_OUTPUT], zb, bl[NUM_OUTPUT:], zb], axis=0)

    sampled = jnp.zeros((CP, blk), jnp.float32)
    banned = jnp.zeros((CP, blk), jnp.float32)
    sample_p = jnp.zeros((CP, blk), jnp.float32)
    ban_p = jnp.zeros((CP, blk), jnp.float32)
    row = jax.lax.broadcasted_iota(jnp.int32, (NC, blk), 0)
    live = (jax.lax.broadcasted_iota(jnp.int32, (CP, blk), 0)
            < NUM_OUTPUT).astype(jnp.float32)
    urow = jax.lax.broadcasted_iota(jnp.uint32, (2 * NUM_OUTPUT, blk), 0)
    ulane = jax.lax.broadcasted_iota(jnp.uint32, (2 * NUM_OUTPUT, blk), 1)
    b0 = (pl.program_id(0) * B_BLK).astype(jnp.uint32)
    flat = (b0 + ulane) * jnp.uint32(2 * NUM_OUTPUT) + urow
    zpad = jnp.zeros((CP - NUM_OUTPUT, blk), jnp.float32)

    for v in range(VOICES):
        sb = jnp.concatenate([sampled, banned], axis=0)
        gi = base_i + jnp.dot(wsb_i, sb, preferred_element_type=jnp.float32)
        gg = base_g + jnp.dot(wsb_g, sb, preferred_element_type=jnp.float32)
        go = base_o + jnp.dot(wsb_o, sb, preferred_element_type=jnp.float32)
        h = jax.nn.sigmoid(go) * jnp.tanh(jax.nn.sigmoid(gi) * jnp.tanh(gg))
        out = jnp.dot(wl32, h, preferred_element_type=jnp.float32) + bl32
        cm = live * (1.0 - sampled) * (1.0 - banned)
        coeff = jnp.concatenate([cm, cm], axis=0)
        p = coeff * jnp.exp(out)
        p = p / jnp.sum(p, axis=0, keepdims=True)
        logits = jnp.where(p > 0, jnp.log(jnp.maximum(p, 1e-30)), -1e9)
        bits = _tf2x32_vec(*_VOICE_KEYS[v], flat)
        fb = (bits >> jnp.uint32(9)) | jnp.uint32(0x3F800000)
        u = jax.lax.bitcast_convert_type(fb, jnp.float32) - 1.0
        un = jnp.maximum(jnp.float32(TINY), u + jnp.float32(TINY))
        g24 = -jnp.log(-jnp.log(un))
        gz = jnp.concatenate([g24[0:NUM_OUTPUT], zpad,
                              g24[NUM_OUTPUT:], zpad], axis=0)
        z = logits + gz
        m = jnp.max(z, axis=0, keepdims=True)
        idx = jnp.min(jnp.where(z == m, row, NC), axis=0, keepdims=True)
        onehot = (row == idx).astype(jnp.float32)
        note = onehot[0:CP, :]
        ban = onehot[CP:, :]
        sample_p = sample_p + note * p[0:CP, :]
        ban_p = ban_p + ban * p[CP:, :]
        sampled = jnp.minimum(sampled + note, 1.0)
        banned = jnp.minimum(banned + ban, 1.0)

    sampled_ref[...] = jnp.transpose(sampled[0:NUM_OUTPUT])
    pn = jnp.transpose(sample_p[0:NUM_OUTPUT])
    pb = jnp.transpose(ban_p[0:NUM_OUTPUT])
    probs_ref[...] = jnp.concatenate([pn, pb], axis=1)


def kernel(x, W_ih, W_hh, b_ih, b_hh, W_lin, b_lin):
    del W_hh
    B = x.shape[1]
    grid = (B // B_BLK,)
    sampled, probs = pl.pallas_call(
        _poly_body,
        grid=grid,
        compiler_params=pltpu.CompilerParams(
            dimension_semantics=("parallel",)),
        in_specs=[
            pl.BlockSpec((B_BLK, AG), lambda i: (i, 0)),
            pl.BlockSpec((4 * H, AG + 2 * NUM_OUTPUT), lambda i: (0, 0)),
            pl.BlockSpec((4 * H, 1), lambda i: (0, 0)),
            pl.BlockSpec((4 * H, 1), lambda i: (0, 0)),
            pl.BlockSpec((2 * NUM_OUTPUT, H), lambda i: (0, 0)),
            pl.BlockSpec((2 * NUM_OUTPUT, 1), lambda i: (0, 0)),
        ],
        out_specs=[
            pl.BlockSpec((B_BLK, NUM_OUTPUT), lambda i: (i, 0)),
            pl.BlockSpec((B_BLK, 2 * NUM_OUTPUT), lambda i: (i, 0)),
        ],
        out_shape=[
            jax.ShapeDtypeStruct((B, NUM_OUTPUT), jnp.float32),
            jax.ShapeDtypeStruct((B, 2 * NUM_OUTPUT), jnp.float32),
        ],
    )(x[0], W_ih, b_ih[:, None], b_hh[:, None], W_lin, b_lin[:, None])
    return (sampled[None], probs[None])


# speedup vs baseline: 2.1492x; 1.0230x over previous
import jax
import jax.numpy as jnp
import numpy as np
from jax.experimental import pallas as pl
from jax.experimental.pallas import tpu as pltpu

NUM_OUTPUT = 12
H = 100
AG = 130
VOICES = 5
CP = 16
NC = 2 * CP
B_BLK = 4096
TINY = float(np.finfo(np.float32).tiny)

_M32 = 0xFFFFFFFF
_ROT_A = (13, 15, 26, 6)
_ROT_B = (17, 29, 16, 24)


def _tf2x32_py(k1, k2, x1, x2):
    ks = (k1, k2, (k1 ^ k2 ^ 0x1BD11BDA) & _M32)
    x = [(x1 + ks[0]) & _M32, (x2 + ks[1]) & _M32]
    sched = ((_ROT_A, ks[1], ks[2], 1), (_ROT_B, ks[2], ks[0], 2),
             (_ROT_A, ks[0], ks[1], 3), (_ROT_B, ks[1], ks[2], 4),
             (_ROT_A, ks[2], ks[0], 5))
    for rots, a0, a1, i in sched:
        for r in rots:
            x[0] = (x[0] + x[1]) & _M32
            x[1] = ((x[1] << r) | (x[1] >> (32 - r))) & _M32
            x[1] = x[0] ^ x[1]
        x[0] = (x[0] + a0) & _M32
        x[1] = (x[1] + a1 + i) & _M32
    return x[0], x[1]


_VOICE_KEYS = tuple(_tf2x32_py(0, 42, 0, v) for v in range(VOICES))


def _tf2x32_vec(k1, k2, x2):
    ks = (jnp.uint32(k1), jnp.uint32(k2),
          jnp.uint32((k1 ^ k2 ^ 0x1BD11BDA) & _M32))
    x0 = jnp.full(x2.shape, ks[0], jnp.uint32)
    x1 = x2 + ks[1]
    sched = ((_ROT_A, ks[1], ks[2], 1), (_ROT_B, ks[2], ks[0], 2),
             (_ROT_A, ks[0], ks[1], 3), (_ROT_B, ks[1], ks[2], 4),
             (_ROT_A, ks[2], ks[0], 5))
    for rots, a0, a1, i in sched:
        for r in rots:
            x0 = x0 + x1
            x1 = (x1 << jnp.uint32(r)) | (x1 >> jnp.uint32(32 - r))
            x1 = x0 ^ x1
        x0 = x0 + a0
        x1 = x1 + a1 + jnp.uint32(i)
    return x0 ^ x1


def _poly_body(x_ref, wih_ref, bih_ref, bhh_ref, wlin_ref, blin_ref,
               sampled_ref, probs_ref):
    blk = x_ref.shape[0]
    wih = wih_ref[...]
    bias = bih_ref[...] + bhh_ref[...]
    wi, wg, wo = wih[0:H], wih[2 * H:3 * H], wih[3 * H:4 * H]
    bi, bg, bo = bias[0:H], bias[2 * H:3 * H], bias[3 * H:4 * H]
    dn = (((1,), (1,)), ((), ()))
    x = x_ref[...]
    base_i = jax.lax.dot_general(wi[:, :AG], x, dn,
                                 preferred_element_type=jnp.float32) + bi
    base_g = jax.lax.dot_general(wg[:, :AG], x, dn,
                                 preferred_element_type=jnp.float32) + bg
    base_o = jax.lax.dot_general(wo[:, :AG], x, dn,
                                 preferred_element_type=jnp.float32) + bo
    zc = jnp.zeros((H, CP - NUM_OUTPUT), jnp.float32)
    wsb_i = jnp.concatenate([wi[:, AG:AG + NUM_OUTPUT], zc,
                             wi[:, AG + NUM_OUTPUT:], zc], axis=1)
    wsb_g = jnp.concatenate([wg[:, AG:AG + NUM_OUTPUT], zc,
                             wg[:, AG + NUM_OUTPUT:], zc], axis=1)
    wsb_o = jnp.concatenate([wo[:, AG:AG + NUM_OUTPUT], zc,
                             wo[:, AG + NUM_OUTPUT:], zc], axis=1)
    wl = wlin_ref[...]
    bl = blin_ref[...]
    zr = jnp.zeros((CP - NUM_OUTPUT, H), jnp.float32)
    zb = jnp.zeros((CP - NUM_OUTPUT, 1), jnp.float32)
    wl32 = jnp.concatenate([wl[0:NUM_OUTPUT], zr, wl[NUM_OUTPUT:], zr], axis=0)
    bl32 = jnp.concatenate([bl[0:NUM_OUTPUT], zb, bl[NUM_OUTPUT:], zb], axis=0)

    sampled = jnp.zeros((CP, blk), jnp.float32)
    banned = jnp.zeros((CP, blk), jnp.float32)
    sample_p = jnp.zeros((CP, blk), jnp.float32)
    ban_p = jnp.zeros((CP, blk), jnp.float32)
    row = jax.lax.broadcasted_iota(jnp.int32, (NC, blk), 0)
    live = (jax.lax.broadcasted_iota(jnp.int32, (CP, blk), 0)
            < NUM_OUTPUT).astype(jnp.float32)
    urow = jax.lax.broadcasted_iota(jnp.uint32, (2 * NUM_OUTPUT, blk), 0)
    ulane = jax.lax.broadcasted_iota(jnp.uint32, (2 * NUM_OUTPUT, blk), 1)
    b0 = (pl.program_id(0) * B_BLK).astype(jnp.uint32)
    flat = (b0 + ulane) * jnp.uint32(2 * NUM_OUTPUT) + urow
    zpad = jnp.zeros((CP - NUM_OUTPUT, blk), jnp.float32)

    for v in range(VOICES):
        sb = jnp.concatenate([sampled, banned], axis=0)
        gi = base_i + jnp.dot(wsb_i, sb, preferred_element_type=jnp.float32)
        gg = base_g + jnp.dot(wsb_g, sb, preferred_element_type=jnp.float32)
        go = base_o + jnp.dot(wsb_o, sb, preferred_element_type=jnp.float32)
        sig_i = 0.5 * jnp.tanh(0.5 * gi) + 0.5
        sig_o = 0.5 * jnp.tanh(0.5 * go) + 0.5
        h = sig_o * jnp.tanh(sig_i * jnp.tanh(gg))
        out = jnp.dot(wl32, h, preferred_element_type=jnp.float32) + bl32
        cm = live * (1.0 - sampled) * (1.0 - banned)
        coeff = jnp.concatenate([cm, cm], axis=0)
        p = coeff * jnp.exp(out)
        p = p / jnp.sum(p, axis=0, keepdims=True)
        logits = jnp.where(p > 0, jnp.log(jnp.maximum(p, 1e-30)), -1e9)
        bits = _tf2x32_vec(*_VOICE_KEYS[v], flat)
        fb = (bits >> jnp.uint32(9)) | jnp.uint32(0x3F800000)
        u = jax.lax.bitcast_convert_type(fb, jnp.float32) - 1.0
        un = jnp.maximum(jnp.float32(TINY), u + jnp.float32(TINY))
        g24 = -jnp.log(-jnp.log(un))
        gz = jnp.concatenate([g24[0:NUM_OUTPUT], zpad,
                              g24[NUM_OUTPUT:], zpad], axis=0)
        z = logits + gz
        m = jnp.max(z, axis=0, keepdims=True)
        idx = jnp.min(jnp.where(z == m, row, NC), axis=0, keepdims=True)
        onehot = (row == idx).astype(jnp.float32)
        note = onehot[0:CP, :]
        ban = onehot[CP:, :]
        sample_p = sample_p + note * p[0:CP, :]
        ban_p = ban_p + ban * p[CP:, :]
        sampled = jnp.minimum(sampled + note, 1.0)
        banned = jnp.minimum(banned + ban, 1.0)

    sampled_ref[...] = jnp.transpose(sampled[0:NUM_OUTPUT])
    pn = jnp.transpose(sample_p[0:NUM_OUTPUT])
    pb = jnp.transpose(ban_p[0:NUM_OUTPUT])
    probs_ref[...] = jnp.concatenate([pn, pb], axis=1)


def kernel(x, W_ih, W_hh, b_ih, b_hh, W_lin, b_lin):
    del W_hh
    B = x.shape[1]
    grid = (B // B_BLK,)
    sampled, probs = pl.pallas_call(
        _poly_body,
        grid=grid,
        compiler_params=pltpu.CompilerParams(
            dimension_semantics=("parallel",)),
        in_specs=[
            pl.BlockSpec((B_BLK, AG), lambda i: (i, 0)),
            pl.BlockSpec((4 * H, AG + 2 * NUM_OUTPUT), lambda i: (0, 0)),
            pl.BlockSpec((4 * H, 1), lambda i: (0, 0)),
            pl.BlockSpec((4 * H, 1), lambda i: (0, 0)),
            pl.BlockSpec((2 * NUM_OUTPUT, H), lambda i: (0, 0)),
            pl.BlockSpec((2 * NUM_OUTPUT, 1), lambda i: (0, 0)),
        ],
        out_specs=[
            pl.BlockSpec((B_BLK, NUM_OUTPUT), lambda i: (i, 0)),
            pl.BlockSpec((B_BLK, 2 * NUM_OUTPUT), lambda i: (i, 0)),
        ],
        out_shape=[
            jax.ShapeDtypeStruct((B, NUM_OUTPUT), jnp.float32),
            jax.ShapeDtypeStruct((B, 2 * NUM_OUTPUT), jnp.float32),
        ],
    )(x[0], W_ih, b_ih[:, None], b_hh[:, None], W_lin, b_lin[:, None])
    return (sampled[None], probs[None])
